# Initial kernel scaffold; baseline (speedup 1.0000x reference)
import jax
import jax.numpy as jnp
from jax.experimental import pallas as pl


def kernel(x, emb, W1, b1, W2, b2):
    raise NotImplementedError("write your pallas kernel here")


# speedup vs baseline: 1.7103x; 1.7103x over previous
import functools

import jax
import jax.numpy as jnp
from jax import lax
from jax.experimental import pallas as pl
from jax.experimental.pallas import tpu as pltpu
from jax.experimental.pallas import tpu_sc as plsc

try:
    _INFO = plsc.get_sparse_core_info()
    _NC = _INFO.num_cores
    _NS = _INFO.num_subcores
except ValueError:
    _NC, _NS = 2, 16
_NW = _NC * _NS
_IDXW = 128


@functools.partial(jax.jit, static_argnums=(2, 3))
def _sc_gather(table, idx2d, n_rows, dim):
    rows_per_w = n_rows // _NW
    chunks = rows_per_w // _IDXW
    mesh = plsc.VectorSubcoreMesh(core_axis_name="c", subcore_axis_name="s")

    @functools.partial(
        pl.kernel,
        mesh=mesh,
        compiler_params=pltpu.CompilerParams(use_tc_tiling_on_sc=False),
        out_type=jax.ShapeDtypeStruct((n_rows, dim), jnp.float32),
        scratch_types=[
            pltpu.VMEM((chunks, _IDXW), jnp.int32),
            pltpu.VMEM((rows_per_w, dim), jnp.float32),
            pltpu.SemaphoreType.DMA,
        ],
    )
    def gather_k(table_hbm, idx_hbm, out_hbm, idx_v, rows_v, sem):
        wid = lax.axis_index("s") * _NC + lax.axis_index("c")
        base = wid * rows_per_w
        pltpu.sync_copy(idx_hbm.at[pl.ds(wid * chunks, chunks)], idx_v)
        copies = []
        for j in range(chunks):
            copies.append(
                pltpu.async_copy(
                    table_hbm.at[idx_v.at[j]],
                    rows_v.at[pl.ds(j * _IDXW, _IDXW)],
                    sem,
                )
            )
        for c in copies:
            c.wait()
        pltpu.sync_copy(rows_v, out_hbm.at[pl.ds(base, rows_per_w)])

    return gather_k(table, idx2d)


def _mlp_body(h_ref, w1_ref, b1_ref, w2_ref, b2_ref, out_ref):
    h = h_ref[...]
    z = jnp.dot(h, w1_ref[...], preferred_element_type=jnp.float32)
    z = jnp.maximum(z + b1_ref[...], 0.0)
    out_ref[...] = (
        jnp.dot(z, w2_ref[...], preferred_element_type=jnp.float32) + b2_ref[...]
    )


@jax.jit
def _tc_mlp(h, W1, b1, W2, b2):
    batch, in_dim = h.shape
    hidden = W1.shape[1]
    out_dim = W2.shape[1]
    block_b = 2048
    grid = (batch // block_b,)
    return pl.pallas_call(
        _mlp_body,
        grid=grid,
        in_specs=[
            pl.BlockSpec((block_b, in_dim), lambda i: (i, 0)),
            pl.BlockSpec((in_dim, hidden), lambda i: (0, 0)),
            pl.BlockSpec((1, hidden), lambda i: (0, 0)),
            pl.BlockSpec((hidden, out_dim), lambda i: (0, 0)),
            pl.BlockSpec((1, out_dim), lambda i: (0, 0)),
        ],
        out_specs=pl.BlockSpec((block_b, out_dim), lambda i: (i, 0)),
        out_shape=jax.ShapeDtypeStruct((batch, out_dim), jnp.float32),
    )(h, W1, b1.reshape(1, -1), W2, b2.reshape(1, -1))


def kernel(x, emb, W1, b1, W2, b2):
    batch = x.shape[0]
    n_rows = batch * 2
    dim = emb.shape[1]
    idx2d = x.reshape(n_rows // _IDXW, _IDXW)
    gathered = _sc_gather(emb, idx2d, n_rows, dim)
    h = gathered.reshape(batch, 2 * dim)
    return _tc_mlp(h, W1, b1, W2, b2)


# speedup vs baseline: 1.7727x; 1.0365x over previous
import functools

import jax
import jax.numpy as jnp
from jax import lax
from jax.experimental import pallas as pl
from jax.experimental.pallas import tpu as pltpu
from jax.experimental.pallas import tpu_sc as plsc

try:
    _INFO = plsc.get_sparse_core_info()
    _NC = _INFO.num_cores
    _NS = _INFO.num_subcores
except ValueError:
    _NC, _NS = 2, 16
_NW = _NC * _NS
_IDXW = 128


@functools.partial(jax.jit, static_argnums=(2, 3))
def _sc_gather(table, idx2d, n_rows, dim):
    rows_per_w = n_rows // _NW
    chunks = rows_per_w // _IDXW
    mesh = plsc.VectorSubcoreMesh(core_axis_name="c", subcore_axis_name="s")

    @functools.partial(
        pl.kernel,
        mesh=mesh,
        compiler_params=pltpu.CompilerParams(use_tc_tiling_on_sc=False),
        out_type=jax.ShapeDtypeStruct((n_rows, dim), jnp.float32),
        scratch_types=[
            pltpu.VMEM((chunks, _IDXW), jnp.int32),
            pltpu.VMEM((rows_per_w, dim), jnp.float32),
            pltpu.SemaphoreType.DMA,
        ],
    )
    def gather_k(table_hbm, idx_hbm, out_hbm, idx_v, rows_v, sem):
        wid = lax.axis_index("s") * _NC + lax.axis_index("c")
        base = wid * rows_per_w
        pltpu.sync_copy(idx_hbm.at[pl.ds(wid * chunks, chunks)], idx_v)
        copies = []
        for j in range(chunks):
            copies.append(
                pltpu.async_copy(
                    table_hbm.at[idx_v.at[j]],
                    rows_v.at[pl.ds(j * _IDXW, _IDXW)],
                    sem,
                )
            )
        for c in copies:
            c.wait()
        pltpu.sync_copy(rows_v, out_hbm.at[pl.ds(base, rows_per_w)])

    return gather_k(table, idx2d)


def _mlp_body(h_ref, w1_ref, b1_ref, w2_ref, b2_ref, out_ref):
    h = h_ref[...]
    z = jnp.dot(h, w1_ref[...], preferred_element_type=jnp.float32)
    z = jnp.maximum(z + b1_ref[...], 0.0)
    out_ref[...] = (
        jnp.dot(z, w2_ref[...], preferred_element_type=jnp.float32) + b2_ref[...]
    )


@jax.jit
def _tc_mlp(h, W1p, b1p, W2p, b2p):
    rows, width = h.shape
    hidden = W1p.shape[1]
    out_dim = W2p.shape[1]
    block_r = 512
    grid = (rows // block_r,)
    return pl.pallas_call(
        _mlp_body,
        grid=grid,
        in_specs=[
            pl.BlockSpec((block_r, width), lambda i: (i, 0)),
            pl.BlockSpec((width, hidden), lambda i: (0, 0)),
            pl.BlockSpec((1, hidden), lambda i: (0, 0)),
            pl.BlockSpec((hidden, out_dim), lambda i: (0, 0)),
            pl.BlockSpec((1, out_dim), lambda i: (0, 0)),
        ],
        out_specs=pl.BlockSpec((block_r, out_dim), lambda i: (i, 0)),
        out_shape=jax.ShapeDtypeStruct((rows, out_dim), jnp.float32),
    )(h, W1p, b1p, W2p, b2p)


def kernel(x, emb, W1, b1, W2, b2):
    batch = x.shape[0]
    n_rows = batch * 2
    dim = emb.shape[1]
    stages = W2.shape[1]
    pack = 128 // (2 * dim)
    idx2d = x.reshape(n_rows // _IDXW, _IDXW)
    gathered = _sc_gather(emb, idx2d, n_rows, dim)
    h = gathered.reshape(batch // pack, 128)
    eye = jnp.eye(pack, dtype=jnp.float32)
    W1p = jnp.kron(eye, W1)
    b1p = jnp.tile(b1, pack).reshape(1, -1)
    W2p = jnp.kron(eye, W2)
    b2p = jnp.tile(b2, pack).reshape(1, -1)
    out = _tc_mlp(h, W1p, b1p, W2p, b2p)
    return out.reshape(batch, stages)
